# Initial kernel scaffold; baseline (speedup 1.0000x reference)
import jax
import jax.numpy as jnp
from jax.experimental import pallas as pl


def kernel(features, norm_coords, coords_int, p_v_dist, wf, wx, b_eff):
    raise NotImplementedError("write your pallas kernel here")


# speedup vs baseline: 187.3154x; 187.3154x over previous
import functools

import jax
import jax.numpy as jnp
from jax.experimental import pallas as pl
from jax.experimental.pallas import tpu as pltpu

_R = 64
_NEG = -1e30


def _zc_kernel(x_ref, w_ref, b_ref, n1_ref, o_ref):
    cout = w_ref.shape[1]
    z = jnp.dot(x_ref[...], w_ref[...],
                preferred_element_type=jnp.float32) + b_ref[...]
    o_ref[:, :cout] = z
    o_ref[:, cout:] = n1_ref[...]


def _proj_kernel(idx_ref, zc_ref, wxc_ref, o_ref, acc_ref, *,
                 ncb, unroll, pb, cout):
    b = pl.program_id(0)
    c = pl.program_id(1)
    nch = pl.num_programs(1)
    lw = cout + 4

    @pl.when(c == 0)
    def _init():
        lane = jax.lax.broadcasted_iota(jnp.int32, (pb, 1, lw), 2)
        acc_ref[...] = jnp.where(lane < cout, _NEG, 0.0)

    maskz = jax.lax.broadcasted_iota(jnp.int32, (1, lw), 1) < cout
    coff = c * ncb

    def body(j, carry):
        k0 = j * unroll
        for uu in range(unroll):
            k = k0 + uu
            i = idx_ref[b, coff + k]
            old = acc_ref[i]
            row = zc_ref[k]
            acc_ref[i] = jnp.where(maskz, jnp.maximum(old, row), old + row)
        return carry

    jax.lax.fori_loop(0, ncb // unroll, body, 0)

    @pl.when(c == nch - 1)
    def _fin():
        a = acc_ref[:, 0, :]
        cnt = jnp.maximum(a[:, cout + 3:cout + 4], 1.0)
        mean = a[:, cout:cout + 3] / cnt
        corr = jnp.dot(mean, wxc_ref[...],
                       preferred_element_type=jnp.float32)
        o_ref[...] = jnp.maximum(a[:, :cout] - corr, 0.0)


def kernel(features, norm_coords, coords_int, p_v_dist, wf, wx, b_eff):
    B, C, Np = features.shape
    N = B * Np
    Cout = wf.shape[1]
    R = _R
    PB = R * R
    LW = Cout + 4

    points = jnp.transpose(features, (0, 2, 1)).reshape(N, C)
    xpyp = p_v_dist[:, 2:4]
    X = jnp.concatenate([points, xpyp, norm_coords], axis=1)
    norm1 = jnp.concatenate(
        [norm_coords, jnp.ones((N, 1), jnp.float32)], axis=1)
    W = jnp.concatenate([wf, wx], axis=0)
    wxc = wx[2:5]
    li = (coords_int[:, 2] * R + coords_int[:, 3]).astype(
        jnp.int32).reshape(B, Np)

    NCA = 1024
    while N % NCA:
        NCA //= 2
    nca = N // NCA
    zc = pl.pallas_call(
        _zc_kernel,
        out_shape=jax.ShapeDtypeStruct((N, LW), jnp.float32),
        grid=(nca,),
        in_specs=[
            pl.BlockSpec((NCA, C + 5), lambda a: (a, 0)),
            pl.BlockSpec((C + 5, Cout), lambda a: (0, 0)),
            pl.BlockSpec((1, Cout), lambda a: (0, 0)),
            pl.BlockSpec((NCA, 4), lambda a: (a, 0)),
        ],
        out_specs=pl.BlockSpec((NCA, LW), lambda a: (a, 0)),
        compiler_params=pltpu.CompilerParams(
            dimension_semantics=("parallel",)),
    )(X, W, b_eff, norm1)

    NCB = 2048
    while Np % NCB:
        NCB //= 2
    nch = Np // NCB
    UNROLL = 8

    kb = functools.partial(_proj_kernel, ncb=NCB, unroll=UNROLL,
                           pb=PB, cout=Cout)
    out2 = pl.pallas_call(
        kb,
        out_shape=jax.ShapeDtypeStruct((B * PB, Cout), jnp.float32),
        grid=(B, nch),
        in_specs=[
            pl.BlockSpec(memory_space=pltpu.SMEM),
            pl.BlockSpec((NCB, 1, LW), lambda b, c: (b * (Np // NCB) + c, 0, 0)),
            pl.BlockSpec((3, Cout), lambda b, c: (0, 0)),
        ],
        out_specs=pl.BlockSpec((PB, Cout), lambda b, c: (b, 0)),
        scratch_shapes=[pltpu.VMEM((PB, 1, LW), jnp.float32)],
        compiler_params=pltpu.CompilerParams(
            dimension_semantics=("parallel", "arbitrary"),
            vmem_limit_bytes=48 * 1024 * 1024,
        ),
    )(li, zc.reshape(N, 1, LW), wxc)

    return out2.reshape(B, R, R, Cout)


# speedup vs baseline: 267.8485x; 1.4299x over previous
import functools

import jax
import jax.numpy as jnp
from jax.experimental import pallas as pl
from jax.experimental.pallas import tpu as pltpu

_R = 64
_NEG = -1e30


def _zc_kernel(x_ref, w_ref, b_ref, n1_ref, o_ref):
    nc = x_ref.shape[0]
    z = jnp.dot(x_ref[...], w_ref[...],
                preferred_element_type=jnp.float32) + b_ref[...]
    o_ref[0:2 * nc:2, :] = z
    o_ref[1:2 * nc:2, 0:4] = n1_ref[...]


def _scat_kernel(idx_ref, zc_ref, o_ref, *, ncb, unroll, pb):
    b = pl.program_id(0)
    c = pl.program_id(1)

    @pl.when(c == 0)
    def _init():
        sub = jax.lax.broadcasted_iota(jnp.int32, (2 * pb, 128), 0)
        o_ref[...] = jnp.where((sub & 1) == 0, _NEG, 0.0)

    mask2 = jax.lax.broadcasted_iota(jnp.int32, (2, 128), 0) == 0
    coff = c * ncb

    def body(j, carry):
        k0 = j * unroll
        for uu in range(unroll):
            k = k0 + uu
            i2 = pl.multiple_of(idx_ref[b, coff + k], 2)
            zn = zc_ref[pl.ds(2 * k, 2), :]
            old = o_ref[pl.ds(i2, 2), :]
            o_ref[pl.ds(i2, 2), :] = jnp.where(
                mask2, jnp.maximum(old, zn), old + zn)
        return carry

    jax.lax.fori_loop(0, ncb // unroll, body, 0)


def _fin_kernel(v_ref, wxc_ref, o_ref):
    cout = o_ref.shape[1]
    x = v_ref[...]
    cnt = jnp.maximum(x[:, cout + 3:cout + 4], 1.0)
    mean = x[:, cout:cout + 3] / cnt
    corr = jnp.dot(mean, wxc_ref[...],
                   preferred_element_type=jnp.float32)
    o_ref[...] = jnp.maximum(x[:, :cout] - corr, 0.0)


def kernel(features, norm_coords, coords_int, p_v_dist, wf, wx, b_eff):
    B, C, Np = features.shape
    N = B * Np
    Cout = wf.shape[1]
    R = _R
    PB = R * R

    points = jnp.transpose(features, (0, 2, 1)).reshape(N, C)
    xpyp = p_v_dist[:, 2:4]
    X = jnp.concatenate([points, xpyp, norm_coords], axis=1)
    norm1 = jnp.concatenate(
        [norm_coords, jnp.ones((N, 1), jnp.float32)], axis=1)
    W = jnp.concatenate([wf, wx], axis=0)
    wxc = wx[2:5]
    li2 = ((coords_int[:, 2] * R + coords_int[:, 3]) * 2).astype(
        jnp.int32).reshape(B, Np)

    NCA = 1024
    while N % NCA:
        NCA //= 2
    zc2 = pl.pallas_call(
        _zc_kernel,
        out_shape=jax.ShapeDtypeStruct((2 * N, Cout), jnp.float32),
        grid=(N // NCA,),
        in_specs=[
            pl.BlockSpec((NCA, C + 5), lambda a: (a, 0)),
            pl.BlockSpec((C + 5, Cout), lambda a: (0, 0)),
            pl.BlockSpec((1, Cout), lambda a: (0, 0)),
            pl.BlockSpec((NCA, 4), lambda a: (a, 0)),
        ],
        out_specs=pl.BlockSpec((2 * NCA, Cout), lambda a: (a, 0)),
        compiler_params=pltpu.CompilerParams(
            dimension_semantics=("parallel",)),
    )(X, W, b_eff, norm1)

    NCB = 2048
    while Np % NCB:
        NCB //= 2
    nch = Np // NCB
    UNROLL = 8

    kb = functools.partial(_scat_kernel, ncb=NCB, unroll=UNROLL, pb=PB)
    acc = pl.pallas_call(
        kb,
        out_shape=jax.ShapeDtypeStruct((B * 2 * PB, Cout), jnp.float32),
        grid=(B, nch),
        in_specs=[
            pl.BlockSpec(memory_space=pltpu.SMEM),
            pl.BlockSpec((2 * NCB, Cout), lambda b, c: (b * (Np // NCB) + c, 0)),
        ],
        out_specs=pl.BlockSpec((2 * PB, Cout), lambda b, c: (b, 0)),
        compiler_params=pltpu.CompilerParams(
            dimension_semantics=("parallel", "arbitrary"),
            vmem_limit_bytes=48 * 1024 * 1024,
        ),
    )(li2, zc2)

    NCC = 2048
    P = B * PB
    while P % NCC:
        NCC //= 2
    out2 = pl.pallas_call(
        _fin_kernel,
        out_shape=jax.ShapeDtypeStruct((P, Cout), jnp.float32),
        grid=(P // NCC,),
        in_specs=[
            pl.BlockSpec((NCC, Cout + 128), lambda a: (a, 0)),
            pl.BlockSpec((3, Cout), lambda a: (0, 0)),
        ],
        out_specs=pl.BlockSpec((NCC, Cout), lambda a: (a, 0)),
        compiler_params=pltpu.CompilerParams(
            dimension_semantics=("parallel",)),
    )(acc.reshape(P, 2 * Cout), wxc)

    return out2.reshape(B, R, R, Cout)


# speedup vs baseline: 309.7221x; 1.1563x over previous
import functools

import jax
import jax.numpy as jnp
from jax.experimental import pallas as pl
from jax.experimental.pallas import tpu as pltpu

_R = 64
_NEG = -1e30


def _zc_kernel(x_ref, w_ref, b_ref, n1_ref, o_ref):
    nc = x_ref.shape[0]
    z = jnp.dot(x_ref[...], w_ref[...],
                preferred_element_type=jnp.float32) + b_ref[...]
    o_ref[0:2 * nc:2, :] = z
    o_ref[1:2 * nc:2, 0:4] = n1_ref[...]


def _scat_kernel(idx_ref, zc_ref, o_ref, a1, a2, a3, *, ncb, unroll, pb):
    b = pl.program_id(0)
    c = pl.program_id(1)
    nch = pl.num_programs(1)
    bufs = (o_ref, a1, a2, a3)

    @pl.when(c == 0)
    def _init():
        for buf in bufs:
            buf[0:2 * pb:2, :] = jnp.full((pb, 128), _NEG, jnp.float32)
            buf[1:2 * pb:2, :] = jnp.zeros((pb, 128), jnp.float32)

    mask2 = jax.lax.broadcasted_iota(jnp.int32, (2, 128), 0) == 0
    coff = c * ncb

    def body(j, carry):
        k0 = j * unroll
        for uu in range(unroll):
            k = k0 + uu
            buf = bufs[uu % 4]
            i2 = pl.multiple_of(idx_ref[b, coff + k], 2)
            zn = zc_ref[pl.ds(2 * k, 2), :]
            old = buf[pl.ds(i2, 2), :]
            buf[pl.ds(i2, 2), :] = jnp.where(
                mask2, jnp.maximum(old, zn), old + zn)
        return carry

    jax.lax.fori_loop(0, ncb // unroll, body, 0)

    @pl.when(c == nch - 1)
    def _merge():
        x0, x1, x2, x3 = (buf[...] for buf in bufs)
        mx = jnp.maximum(jnp.maximum(x0, x1), jnp.maximum(x2, x3))
        sm = (x0 + x1) + (x2 + x3)
        sub = jax.lax.broadcasted_iota(jnp.int32, (2 * pb, 128), 0)
        o_ref[...] = jnp.where((sub & 1) == 0, mx, sm)


def _fin_kernel(v_ref, wxc_ref, o_ref):
    cout = o_ref.shape[1]
    x = v_ref[...]
    cnt = jnp.maximum(x[:, cout + 3:cout + 4], 1.0)
    mean = x[:, cout:cout + 3] / cnt
    corr = jnp.dot(mean, wxc_ref[...],
                   preferred_element_type=jnp.float32)
    o_ref[...] = jnp.maximum(x[:, :cout] - corr, 0.0)


def kernel(features, norm_coords, coords_int, p_v_dist, wf, wx, b_eff):
    B, C, Np = features.shape
    N = B * Np
    Cout = wf.shape[1]
    R = _R
    PB = R * R

    points = jnp.transpose(features, (0, 2, 1)).reshape(N, C)
    xpyp = p_v_dist[:, 2:4]
    X = jnp.concatenate([points, xpyp, norm_coords], axis=1)
    norm1 = jnp.concatenate(
        [norm_coords, jnp.ones((N, 1), jnp.float32)], axis=1)
    W = jnp.concatenate([wf, wx], axis=0)
    wxc = wx[2:5]
    li2 = ((coords_int[:, 2] * R + coords_int[:, 3]) * 2).astype(
        jnp.int32).reshape(B, Np)

    NCA = 1024
    while N % NCA:
        NCA //= 2
    zc2 = pl.pallas_call(
        _zc_kernel,
        out_shape=jax.ShapeDtypeStruct((2 * N, Cout), jnp.float32),
        grid=(N // NCA,),
        in_specs=[
            pl.BlockSpec((NCA, C + 5), lambda a: (a, 0)),
            pl.BlockSpec((C + 5, Cout), lambda a: (0, 0)),
            pl.BlockSpec((1, Cout), lambda a: (0, 0)),
            pl.BlockSpec((NCA, 4), lambda a: (a, 0)),
        ],
        out_specs=pl.BlockSpec((2 * NCA, Cout), lambda a: (a, 0)),
        compiler_params=pltpu.CompilerParams(
            dimension_semantics=("parallel",)),
    )(X, W, b_eff, norm1)

    NCB = 2048
    while Np % NCB:
        NCB //= 2
    nch = Np // NCB
    UNROLL = 8

    kb = functools.partial(_scat_kernel, ncb=NCB, unroll=UNROLL, pb=PB)
    acc = pl.pallas_call(
        kb,
        out_shape=jax.ShapeDtypeStruct((B * 2 * PB, Cout), jnp.float32),
        grid=(B, nch),
        in_specs=[
            pl.BlockSpec(memory_space=pltpu.SMEM),
            pl.BlockSpec((2 * NCB, Cout), lambda b, c: (b * (Np // NCB) + c, 0)),
        ],
        out_specs=pl.BlockSpec((2 * PB, Cout), lambda b, c: (b, 0)),
        scratch_shapes=[pltpu.VMEM((2 * PB, Cout), jnp.float32)
                        for _ in range(3)],
        compiler_params=pltpu.CompilerParams(
            dimension_semantics=("parallel", "arbitrary"),
            vmem_limit_bytes=48 * 1024 * 1024,
        ),
    )(li2, zc2)

    NCC = 2048
    P = B * PB
    while P % NCC:
        NCC //= 2
    out2 = pl.pallas_call(
        _fin_kernel,
        out_shape=jax.ShapeDtypeStruct((P, Cout), jnp.float32),
        grid=(P // NCC,),
        in_specs=[
            pl.BlockSpec((NCC, Cout + 128), lambda a: (a, 0)),
            pl.BlockSpec((3, Cout), lambda a: (0, 0)),
        ],
        out_specs=pl.BlockSpec((NCC, Cout), lambda a: (a, 0)),
        compiler_params=pltpu.CompilerParams(
            dimension_semantics=("parallel",)),
    )(acc.reshape(P, 2 * Cout), wxc)

    return out2.reshape(B, R, R, Cout)


# speedup vs baseline: 335.3245x; 1.0827x over previous
import functools

import jax
import jax.numpy as jnp
from jax.experimental import pallas as pl
from jax.experimental.pallas import tpu as pltpu

_R = 64
_NEG = -1e30


def _zc_kernel(x_ref, w_ref, b_ref, n1_ref, o_ref):
    nc = x_ref.shape[0]
    z = jnp.dot(x_ref[...], w_ref[...],
                preferred_element_type=jnp.float32) + b_ref[...]
    o_ref[0:2 * nc:2, :] = z
    o_ref[1:2 * nc:2, 0:4] = n1_ref[...]


def _scat_kernel(idx_ref, zc_ref, o_ref, a1, a2, a3, *, ncb, unroll, pb):
    b = pl.program_id(0)
    c = pl.program_id(1)
    nch = pl.num_programs(1)
    bufs = (o_ref, a1, a2, a3)

    @pl.when(c == 0)
    def _init():
        for buf in bufs:
            buf[0:2 * pb:2, :] = jnp.full((pb, 128), _NEG, jnp.float32)
            buf[1:2 * pb:2, :] = jnp.zeros((pb, 128), jnp.float32)

    mask2 = jax.lax.broadcasted_iota(jnp.int32, (2, 128), 0) == 0
    coff = c * ncb

    def body(j, carry):
        k0 = j * unroll
        for g in range(unroll // 4):
            ks = [k0 + 4 * g + t for t in range(4)]
            i2s = [pl.multiple_of(idx_ref[b, coff + k], 2) for k in ks]
            zns = [zc_ref[pl.ds(2 * k, 2), :] for k in ks]
            olds = [bufs[t][pl.ds(i2s[t], 2), :] for t in range(4)]
            news = [jnp.where(mask2, jnp.maximum(olds[t], zns[t]),
                              olds[t] + zns[t]) for t in range(4)]
            for t in range(4):
                bufs[t][pl.ds(i2s[t], 2), :] = news[t]
        return carry

    jax.lax.fori_loop(0, ncb // unroll, body, 0)

    @pl.when(c == nch - 1)
    def _merge():
        x0, x1, x2, x3 = (buf[...] for buf in bufs)
        mx = jnp.maximum(jnp.maximum(x0, x1), jnp.maximum(x2, x3))
        sm = (x0 + x1) + (x2 + x3)
        sub = jax.lax.broadcasted_iota(jnp.int32, (2 * pb, 128), 0)
        o_ref[...] = jnp.where((sub & 1) == 0, mx, sm)


def _fin_kernel(v_ref, wxc_ref, o_ref):
    cout = o_ref.shape[1]
    x = v_ref[...]
    cnt = jnp.maximum(x[:, cout + 3:cout + 4], 1.0)
    mean = x[:, cout:cout + 3] / cnt
    corr = jnp.dot(mean, wxc_ref[...],
                   preferred_element_type=jnp.float32)
    o_ref[...] = jnp.maximum(x[:, :cout] - corr, 0.0)


def kernel(features, norm_coords, coords_int, p_v_dist, wf, wx, b_eff):
    B, C, Np = features.shape
    N = B * Np
    Cout = wf.shape[1]
    R = _R
    PB = R * R

    points = jnp.transpose(features, (0, 2, 1)).reshape(N, C)
    xpyp = p_v_dist[:, 2:4]
    X = jnp.concatenate([points, xpyp, norm_coords], axis=1)
    norm1 = jnp.concatenate(
        [norm_coords, jnp.ones((N, 1), jnp.float32)], axis=1)
    W = jnp.concatenate([wf, wx], axis=0)
    wxc = wx[2:5]
    li2 = ((coords_int[:, 2] * R + coords_int[:, 3]) * 2).astype(
        jnp.int32).reshape(B, Np)

    NCA = 1024
    while N % NCA:
        NCA //= 2
    zc2 = pl.pallas_call(
        _zc_kernel,
        out_shape=jax.ShapeDtypeStruct((2 * N, Cout), jnp.float32),
        grid=(N // NCA,),
        in_specs=[
            pl.BlockSpec((NCA, C + 5), lambda a: (a, 0)),
            pl.BlockSpec((C + 5, Cout), lambda a: (0, 0)),
            pl.BlockSpec((1, Cout), lambda a: (0, 0)),
            pl.BlockSpec((NCA, 4), lambda a: (a, 0)),
        ],
        out_specs=pl.BlockSpec((2 * NCA, Cout), lambda a: (a, 0)),
        compiler_params=pltpu.CompilerParams(
            dimension_semantics=("parallel",)),
    )(X, W, b_eff, norm1)

    NCB = 2048
    while Np % NCB:
        NCB //= 2
    nch = Np // NCB
    UNROLL = 16

    kb = functools.partial(_scat_kernel, ncb=NCB, unroll=UNROLL, pb=PB)
    acc = pl.pallas_call(
        kb,
        out_shape=jax.ShapeDtypeStruct((B * 2 * PB, Cout), jnp.float32),
        grid=(B, nch),
        in_specs=[
            pl.BlockSpec(memory_space=pltpu.SMEM),
            pl.BlockSpec((2 * NCB, Cout), lambda b, c: (b * (Np // NCB) + c, 0)),
        ],
        out_specs=pl.BlockSpec((2 * PB, Cout), lambda b, c: (b, 0)),
        scratch_shapes=[pltpu.VMEM((2 * PB, Cout), jnp.float32)
                        for _ in range(3)],
        compiler_params=pltpu.CompilerParams(
            dimension_semantics=("parallel", "arbitrary"),
            vmem_limit_bytes=48 * 1024 * 1024,
        ),
    )(li2, zc2)

    NCC = 2048
    P = B * PB
    while P % NCC:
        NCC //= 2
    out2 = pl.pallas_call(
        _fin_kernel,
        out_shape=jax.ShapeDtypeStruct((P, Cout), jnp.float32),
        grid=(P // NCC,),
        in_specs=[
            pl.BlockSpec((NCC, Cout + 128), lambda a: (a, 0)),
            pl.BlockSpec((3, Cout), lambda a: (0, 0)),
        ],
        out_specs=pl.BlockSpec((NCC, Cout), lambda a: (a, 0)),
        compiler_params=pltpu.CompilerParams(
            dimension_semantics=("parallel",)),
    )(acc.reshape(P, 2 * Cout), wxc)

    return out2.reshape(B, R, R, Cout)


# speedup vs baseline: 424.1642x; 1.2649x over previous
import functools

import jax
import jax.numpy as jnp
from jax.experimental import pallas as pl
from jax.experimental.pallas import tpu as pltpu

_R = 64
_NEG = -1e30


def _proj_kernel(idx_ref, x_ref, w_ref, b_ref, n1_ref, wxc_ref, o_ref,
                 zbuf, a0, a1, a2, a3, *, ncb, unroll, pb):
    b = pl.program_id(0)
    c = pl.program_id(1)
    nch = pl.num_programs(1)
    bufs = (a0, a1, a2, a3)

    @pl.when(c == 0)
    def _init():
        for buf in bufs:
            buf[0:2 * pb:2, :] = jnp.full((pb, 128), _NEG, jnp.float32)
            buf[1:2 * pb:2, :] = jnp.zeros((pb, 128), jnp.float32)

    z = jnp.dot(x_ref[...], w_ref[...],
                preferred_element_type=jnp.float32) + b_ref[...]
    zbuf[0:2 * ncb:2, :] = z
    zbuf[1:2 * ncb:2, 0:4] = n1_ref[...]

    mask2 = jax.lax.broadcasted_iota(jnp.int32, (2, 128), 0) == 0
    coff = c * ncb

    def body(j, carry):
        k0 = j * unroll
        for g in range(unroll // 4):
            ks = [k0 + 4 * g + t for t in range(4)]
            i2s = [pl.multiple_of(idx_ref[b, coff + k], 2) for k in ks]
            zns = [zbuf[pl.ds(2 * k, 2), :] for k in ks]
            olds = [bufs[t][pl.ds(i2s[t], 2), :] for t in range(4)]
            news = [jnp.where(mask2, jnp.maximum(olds[t], zns[t]),
                              olds[t] + zns[t]) for t in range(4)]
            for t in range(4):
                bufs[t][pl.ds(i2s[t], 2), :] = news[t]
        return carry

    jax.lax.fori_loop(0, ncb // unroll, body, 0)

    @pl.when(c == nch - 1)
    def _fin():
        m01 = jnp.maximum(a0[0:2 * pb:2, :], a1[0:2 * pb:2, :])
        m23 = jnp.maximum(a2[0:2 * pb:2, :], a3[0:2 * pb:2, :])
        m = jnp.maximum(m01, m23)
        s01 = a0[1:2 * pb:2, 0:4] + a1[1:2 * pb:2, 0:4]
        s23 = a2[1:2 * pb:2, 0:4] + a3[1:2 * pb:2, 0:4]
        s = s01 + s23
        cnt = jnp.maximum(s[:, 3:4], 1.0)
        mean = s[:, 0:3] / cnt
        corr = jnp.dot(mean, wxc_ref[...],
                       preferred_element_type=jnp.float32)
        o_ref[...] = jnp.maximum(m - corr, 0.0)


def kernel(features, norm_coords, coords_int, p_v_dist, wf, wx, b_eff):
    B, C, Np = features.shape
    N = B * Np
    Cout = wf.shape[1]
    R = _R
    PB = R * R

    points = jnp.transpose(features, (0, 2, 1)).reshape(N, C)
    xpyp = p_v_dist[:, 2:4]
    X = jnp.concatenate([points, xpyp, norm_coords], axis=1)
    norm1 = jnp.concatenate(
        [norm_coords, jnp.ones((N, 1), jnp.float32)], axis=1)
    W = jnp.concatenate([wf, wx], axis=0)
    wxc = wx[2:5]
    li2 = ((coords_int[:, 2] * R + coords_int[:, 3]) * 2).astype(
        jnp.int32).reshape(B, Np)

    NCB = 2048
    while Np % NCB:
        NCB //= 2
    nch = Np // NCB
    UNROLL = 16

    kb = functools.partial(_proj_kernel, ncb=NCB, unroll=UNROLL, pb=PB)
    out2 = pl.pallas_call(
        kb,
        out_shape=jax.ShapeDtypeStruct((B * PB, Cout), jnp.float32),
        grid=(B, nch),
        in_specs=[
            pl.BlockSpec(memory_space=pltpu.SMEM),
            pl.BlockSpec((NCB, C + 5), lambda b, c: (b * (Np // NCB) + c, 0)),
            pl.BlockSpec((C + 5, Cout), lambda b, c: (0, 0)),
            pl.BlockSpec((1, Cout), lambda b, c: (0, 0)),
            pl.BlockSpec((NCB, 4), lambda b, c: (b * (Np // NCB) + c, 0)),
            pl.BlockSpec((3, Cout), lambda b, c: (0, 0)),
        ],
        out_specs=pl.BlockSpec((PB, Cout), lambda b, c: (b, 0)),
        scratch_shapes=[pltpu.VMEM((2 * NCB, Cout), jnp.float32)] +
                       [pltpu.VMEM((2 * PB, Cout), jnp.float32)
                        for _ in range(4)],
        compiler_params=pltpu.CompilerParams(
            dimension_semantics=("parallel", "arbitrary"),
            vmem_limit_bytes=48 * 1024 * 1024,
        ),
    )(li2, X, W, b_eff, norm1, wxc)

    return out2.reshape(B, R, R, Cout)


# speedup vs baseline: 526.0758x; 1.2403x over previous
import functools

import jax
import jax.numpy as jnp
from jax.experimental import pallas as pl
from jax.experimental.pallas import tpu as pltpu

_R = 64
_NEG = -1e30


def _proj_kernel(idx_ref, f_ref, pv_ref, n_ref, wf_ref, wx01_ref, wxc_ref,
                 b_ref, o_ref, zbuf, a0, a1, a2, a3, *, ncb, unroll, pb):
    b = pl.program_id(0)
    c = pl.program_id(1)
    nch = pl.num_programs(1)
    bufs = (a0, a1, a2, a3)

    @pl.when(c == 0)
    def _init():
        for buf in bufs:
            buf[0:2 * pb:2, :] = jnp.full((pb, 128), _NEG, jnp.float32)
            buf[1:2 * pb:2, :] = jnp.zeros((pb, 128), jnp.float32)

    norm = n_ref[...]
    xpyp = pv_ref[...][:, 2:4]
    z = (jax.lax.dot_general(f_ref[0], wf_ref[...], (((0,), (0,)), ((), ())),
                             preferred_element_type=jnp.float32)
         + jnp.dot(xpyp, wx01_ref[...],
                   preferred_element_type=jnp.float32)
         + jnp.dot(norm, wxc_ref[...],
                   preferred_element_type=jnp.float32)
         + b_ref[...])
    zbuf[0:2 * ncb:2, :] = z
    zbuf[1:2 * ncb:2, 0:4] = jnp.concatenate(
        [norm, jnp.ones((ncb, 1), jnp.float32)], axis=1)

    mask2 = jax.lax.broadcasted_iota(jnp.int32, (2, 128), 0) == 0
    coff = c * ncb

    def body(j, carry):
        k0 = j * unroll
        for g in range(unroll // 4):
            ks = [k0 + 4 * g + t for t in range(4)]
            i2s = [pl.multiple_of(idx_ref[b, coff + k], 2) for k in ks]
            zns = [zbuf[pl.ds(2 * k, 2), :] for k in ks]
            olds = [bufs[t][pl.ds(i2s[t], 2), :] for t in range(4)]
            news = [jnp.where(mask2, jnp.maximum(olds[t], zns[t]),
                              olds[t] + zns[t]) for t in range(4)]
            for t in range(4):
                bufs[t][pl.ds(i2s[t], 2), :] = news[t]
        return carry

    jax.lax.fori_loop(0, ncb // unroll, body, 0)

    @pl.when(c == nch - 1)
    def _fin():
        m01 = jnp.maximum(a0[0:2 * pb:2, :], a1[0:2 * pb:2, :])
        m23 = jnp.maximum(a2[0:2 * pb:2, :], a3[0:2 * pb:2, :])
        m = jnp.maximum(m01, m23)
        s01 = a0[1:2 * pb:2, 0:4] + a1[1:2 * pb:2, 0:4]
        s23 = a2[1:2 * pb:2, 0:4] + a3[1:2 * pb:2, 0:4]
        s = s01 + s23
        cnt = jnp.maximum(s[:, 3:4], 1.0)
        mean = s[:, 0:3] / cnt
        corr = jnp.dot(mean, wxc_ref[...],
                       preferred_element_type=jnp.float32)
        o_ref[...] = jnp.maximum(m - corr, 0.0)


def kernel(features, norm_coords, coords_int, p_v_dist, wf, wx, b_eff):
    B, C, Np = features.shape
    N = B * Np
    Cout = wf.shape[1]
    R = _R
    PB = R * R

    wx01 = wx[0:2]
    wxc = wx[2:5]
    li2 = ((coords_int[:, 2] * R + coords_int[:, 3]) * 2).astype(
        jnp.int32).reshape(B, Np)

    NCB = 2048
    while Np % NCB:
        NCB //= 2
    nch = Np // NCB
    UNROLL = 16

    kb = functools.partial(_proj_kernel, ncb=NCB, unroll=UNROLL, pb=PB)
    out2 = pl.pallas_call(
        kb,
        out_shape=jax.ShapeDtypeStruct((B * PB, Cout), jnp.float32),
        grid=(B, nch),
        in_specs=[
            pl.BlockSpec(memory_space=pltpu.SMEM),
            pl.BlockSpec((1, C, NCB), lambda b, c: (b, 0, c)),
            pl.BlockSpec((NCB, 4), lambda b, c: (b * (Np // NCB) + c, 0)),
            pl.BlockSpec((NCB, 3), lambda b, c: (b * (Np // NCB) + c, 0)),
            pl.BlockSpec((C, Cout), lambda b, c: (0, 0)),
            pl.BlockSpec((2, Cout), lambda b, c: (0, 0)),
            pl.BlockSpec((3, Cout), lambda b, c: (0, 0)),
            pl.BlockSpec((1, Cout), lambda b, c: (0, 0)),
        ],
        out_specs=pl.BlockSpec((PB, Cout), lambda b, c: (b, 0)),
        scratch_shapes=[pltpu.VMEM((2 * NCB, Cout), jnp.float32)] +
                       [pltpu.VMEM((2 * PB, Cout), jnp.float32)
                        for _ in range(4)],
        compiler_params=pltpu.CompilerParams(
            dimension_semantics=("parallel", "arbitrary"),
            vmem_limit_bytes=48 * 1024 * 1024,
        ),
    )(li2, features, p_v_dist, norm_coords, wf, wx01, wxc, b_eff)

    return out2.reshape(B, R, R, Cout)


# speedup vs baseline: 539.8118x; 1.0261x over previous
import functools

import jax
import jax.numpy as jnp
from jax.experimental import pallas as pl
from jax.experimental.pallas import tpu as pltpu

_R = 64
_NEG = -1e30


def _proj_kernel(idx_ref, f_ref, pv_ref, n_ref, wf_ref, wx01_ref, wxc_ref,
                 b_ref, o_ref, zbuf, a0, a1, a2, a3, a4, a5, a6, a7, *,
                 ncb, unroll, pb):
    b = pl.program_id(0)
    c = pl.program_id(1)
    nch = pl.num_programs(1)
    bufs = (a0, a1, a2, a3, a4, a5, a6, a7)

    @pl.when(c == 0)
    def _init():
        for buf in bufs:
            buf[0:2 * pb:2, :] = jnp.full((pb, 128), _NEG, jnp.float32)
            buf[1:2 * pb:2, :] = jnp.zeros((pb, 128), jnp.float32)

    norm = n_ref[...]
    xpyp = pv_ref[...][:, 2:4]
    z = (jax.lax.dot_general(f_ref[0], wf_ref[...], (((0,), (0,)), ((), ())),
                             preferred_element_type=jnp.float32)
         + jnp.dot(xpyp, wx01_ref[...],
                   preferred_element_type=jnp.float32)
         + jnp.dot(norm, wxc_ref[...],
                   preferred_element_type=jnp.float32)
         + b_ref[...])
    zbuf[0:2 * ncb:2, :] = z
    zbuf[1:2 * ncb:2, 0:4] = jnp.concatenate(
        [norm, jnp.ones((ncb, 1), jnp.float32)], axis=1)

    mask2 = jax.lax.broadcasted_iota(jnp.int32, (2, 128), 0) == 0
    coff = c * ncb

    def body(j, carry):
        k0 = j * unroll
        for g in range(unroll // 8):
            ks = [k0 + 8 * g + t for t in range(8)]
            i2s = [pl.multiple_of(idx_ref[b, coff + k], 2) for k in ks]
            zns = [zbuf[pl.ds(2 * k, 2), :] for k in ks]
            olds = [bufs[t][pl.ds(i2s[t], 2), :] for t in range(8)]
            news = [jnp.where(mask2, jnp.maximum(olds[t], zns[t]),
                              olds[t] + zns[t]) for t in range(8)]
            for t in range(8):
                bufs[t][pl.ds(i2s[t], 2), :] = news[t]
        return carry

    jax.lax.fori_loop(0, ncb // unroll, body, 0)

    @pl.when(c == nch - 1)
    def _fin():
        ms = [buf[0:2 * pb:2, :] for buf in bufs]
        while len(ms) > 1:
            ms = [jnp.maximum(ms[i], ms[i + 1]) for i in range(0, len(ms), 2)]
        m = ms[0]
        ss = [buf[1:2 * pb:2, 0:4] for buf in bufs]
        while len(ss) > 1:
            ss = [ss[i] + ss[i + 1] for i in range(0, len(ss), 2)]
        s = ss[0]
        cnt = jnp.maximum(s[:, 3:4], 1.0)
        mean = s[:, 0:3] / cnt
        corr = jnp.dot(mean, wxc_ref[...],
                       preferred_element_type=jnp.float32)
        o_ref[...] = jnp.maximum(m - corr, 0.0)


def kernel(features, norm_coords, coords_int, p_v_dist, wf, wx, b_eff):
    B, C, Np = features.shape
    N = B * Np
    Cout = wf.shape[1]
    R = _R
    PB = R * R

    wx01 = wx[0:2]
    wxc = wx[2:5]
    li2 = ((coords_int[:, 2] * R + coords_int[:, 3]) * 2).astype(
        jnp.int32).reshape(B, Np)

    NCB = 2048
    while Np % NCB:
        NCB //= 2
    nch = Np // NCB
    UNROLL = 32

    kb = functools.partial(_proj_kernel, ncb=NCB, unroll=UNROLL, pb=PB)
    out2 = pl.pallas_call(
        kb,
        out_shape=jax.ShapeDtypeStruct((B * PB, Cout), jnp.float32),
        grid=(B, nch),
        in_specs=[
            pl.BlockSpec(memory_space=pltpu.SMEM),
            pl.BlockSpec((1, C, NCB), lambda b, c: (b, 0, c)),
            pl.BlockSpec((NCB, 4), lambda b, c: (b * (Np // NCB) + c, 0)),
            pl.BlockSpec((NCB, 3), lambda b, c: (b * (Np // NCB) + c, 0)),
            pl.BlockSpec((C, Cout), lambda b, c: (0, 0)),
            pl.BlockSpec((2, Cout), lambda b, c: (0, 0)),
            pl.BlockSpec((3, Cout), lambda b, c: (0, 0)),
            pl.BlockSpec((1, Cout), lambda b, c: (0, 0)),
        ],
        out_specs=pl.BlockSpec((PB, Cout), lambda b, c: (b, 0)),
        scratch_shapes=[pltpu.VMEM((2 * NCB, Cout), jnp.float32)] +
                       [pltpu.VMEM((2 * PB, Cout), jnp.float32)
                        for _ in range(8)],
        compiler_params=pltpu.CompilerParams(
            dimension_semantics=("parallel", "arbitrary"),
            vmem_limit_bytes=48 * 1024 * 1024,
        ),
    )(li2, features, p_v_dist, norm_coords, wf, wx01, wxc, b_eff)

    return out2.reshape(B, R, R, Cout)
